# baseline (speedup 1.0000x reference)
import jax
import jax.numpy as jnp
from jax.experimental import pallas as pl
from jax.experimental.pallas import tpu as pltpu
from functools import partial

_DT = 1.0 / 10.0


def _drive_kernel(x_ref, wx_ref, b_ref, d_out_ref, *, bba, tdim, hdim):
    for i in range(bba):
        xs = x_ref[:, i, :]
        d = jax.lax.dot_general(
            xs, wx_ref[...],
            dimension_numbers=(((0,), (0,)), ((), ())),
            preferred_element_type=jnp.float32)
        d_out_ref[i] = (d + b_ref[0, :]).astype(jnp.bfloat16)


def _scan_kernel(d_ref, noise_ref, wh_ref, wy_ref, ah0_ref,
                 h_out_ref, y_out_ref, ah_scr, h_scr, drive_scr,
                 *, bb, tt, hdim):
    t_blk = pl.program_id(1)

    @pl.when(t_blk == 0)
    def _init():
        ah0 = jnp.broadcast_to(ah0_ref[0, :], (bb, hdim))
        ah_scr[...] = ah0
        h_scr[...] = jnp.maximum(jnp.tanh(ah0), 0.0)

    drive_scr[...] = d_ref[...].astype(jnp.float32)

    ah = ah_scr[...]
    hcur = h_scr[...]
    for t in range(tt):
        rec = jnp.dot(hcur, wh_ref[...], preferred_element_type=jnp.float32)
        ah = (1.0 - _DT) * ah + rec + drive_scr[:, t, :]
        hcur = jnp.maximum(jnp.tanh(ah), 0.0) + noise_ref[:, t, :]
        h_out_ref[:, t, :] = hcur
    ah_scr[...] = ah
    h_scr[...] = hcur

    hs = h_out_ref[...].reshape(bb * tt, hdim)
    y = jnp.dot(hs, wy_ref[...], preferred_element_type=jnp.float32)
    y_out_ref[...] = y.reshape(bb, tt, y_out_ref.shape[-1])


@partial(jax.jit, static_argnames=("interpret",))
def kernel(x, noise, W_x_ah, b_ah, W_h_ah, W_h_y, ah0, interpret=False):
    B, T, DIN = x.shape
    H = W_h_ah.shape[0]
    DOUT = W_h_y.shape[0]

    BBA = 16
    BB = 256
    TT = 8

    wx = (_DT * W_x_ah).T
    wh = (_DT * W_h_ah).T
    bs = (_DT * b_ah).reshape(1, H)
    wy = W_h_y.T
    ah0r = ah0.reshape(1, H)

    xT = jnp.transpose(x, (2, 0, 1))

    drive = pl.pallas_call(
        partial(_drive_kernel, bba=BBA, tdim=T, hdim=H),
        grid=(B // BBA,),
        in_specs=[
            pl.BlockSpec((DIN, BBA, T), lambda b: (0, b, 0)),
            pl.BlockSpec((DIN, H), lambda b: (0, 0)),
            pl.BlockSpec((1, H), lambda b: (0, 0)),
        ],
        out_specs=pl.BlockSpec((BBA, T, H), lambda b: (b, 0, 0)),
        out_shape=jax.ShapeDtypeStruct((B, T, H), jnp.bfloat16),
        compiler_params=pltpu.CompilerParams(
            dimension_semantics=("parallel",),
            vmem_limit_bytes=48 * 1024 * 1024,
        ),
        name="ctrnn_drive",
        interpret=interpret,
    )(xT, wx, bs)

    out_shape = (
        jax.ShapeDtypeStruct((B, T, H), jnp.float32),
        jax.ShapeDtypeStruct((B, T, DOUT), jnp.float32),
    )

    hstore, output = pl.pallas_call(
        partial(_scan_kernel, bb=BB, tt=TT, hdim=H),
        grid=(B // BB, T // TT),
        in_specs=[
            pl.BlockSpec((BB, TT, H), lambda b, t: (b, t, 0)),
            pl.BlockSpec((BB, TT, H), lambda b, t: (b, t, 0)),
            pl.BlockSpec((H, H), lambda b, t: (0, 0)),
            pl.BlockSpec((H, DOUT), lambda b, t: (0, 0)),
            pl.BlockSpec((1, H), lambda b, t: (0, 0)),
        ],
        out_specs=[
            pl.BlockSpec((BB, TT, H), lambda b, t: (b, t, 0)),
            pl.BlockSpec((BB, TT, DOUT), lambda b, t: (b, t, 0)),
        ],
        out_shape=out_shape,
        scratch_shapes=[
            pltpu.VMEM((BB, H), jnp.float32),
            pltpu.VMEM((BB, H), jnp.float32),
            pltpu.VMEM((BB, TT, H), jnp.float32),
        ],
        compiler_params=pltpu.CompilerParams(
            dimension_semantics=("parallel", "arbitrary"),
            vmem_limit_bytes=56 * 1024 * 1024,
        ),
        name="ctrnn_scan",
        interpret=interpret,
    )(drive, noise, wh, wy, ah0r)

    return output, hstore


# speedup vs baseline: 1.0958x; 1.0958x over previous
import jax
import jax.numpy as jnp
from jax.experimental import pallas as pl
from jax.experimental.pallas import tpu as pltpu
from functools import partial

_DT = 1.0 / 10.0


def _drive_kernel(x_ref, wx_ref, b_ref, d_out_ref, *, bba, tdim, hdim):
    for i in range(bba):
        xs = x_ref[:, i, :]
        d = jax.lax.dot_general(
            xs, wx_ref[...],
            dimension_numbers=(((0,), (0,)), ((), ())),
            preferred_element_type=jnp.float32)
        d_out_ref[i] = (d + b_ref[0, :]).astype(jnp.bfloat16)


def _scan_kernel(d_ref, noise_ref, wh_ref, wy_ref, ah0_ref,
                 h_out_ref, y_out_ref, ah_scr, h_scr, drive_scr,
                 *, bb, tt, hdim):
    t_blk = pl.program_id(1)

    @pl.when(t_blk == 0)
    def _init():
        ah0 = jnp.broadcast_to(ah0_ref[0, :], (bb, hdim))
        ah_scr[...] = ah0
        h_scr[...] = jnp.maximum(jnp.tanh(ah0), 0.0)

    drive_scr[...] = d_ref[...].astype(jnp.float32)

    ah = ah_scr[...]
    hcur = h_scr[...]
    for t in range(tt):
        rec = jnp.dot(hcur, wh_ref[...], preferred_element_type=jnp.float32)
        ah = (1.0 - _DT) * ah + rec + drive_scr[:, t, :]
        hcur = jnp.maximum(jnp.tanh(ah), 0.0) + noise_ref[:, t, :]
        h_out_ref[:, t, :] = hcur
    ah_scr[...] = ah
    h_scr[...] = hcur

    hs = h_out_ref[...].reshape(bb * tt, hdim)
    y = jnp.dot(hs, wy_ref[...], preferred_element_type=jnp.float32)
    y_out_ref[...] = y.reshape(bb, tt, y_out_ref.shape[-1])


@partial(jax.jit, static_argnames=("interpret",))
def kernel(x, noise, W_x_ah, b_ah, W_h_ah, W_h_y, ah0, interpret=False):
    B, T, DIN = x.shape
    H = W_h_ah.shape[0]
    DOUT = W_h_y.shape[0]

    BBA = 8
    BB = 256
    TT = 8

    wx = (_DT * W_x_ah).T
    wh = (_DT * W_h_ah).T
    bs = (_DT * b_ah).reshape(1, H)
    wy = W_h_y.T
    ah0r = ah0.reshape(1, H)

    xT = jnp.transpose(x, (2, 0, 1))

    drive = pl.pallas_call(
        partial(_drive_kernel, bba=BBA, tdim=T, hdim=H),
        grid=(B // BBA,),
        in_specs=[
            pl.BlockSpec((DIN, BBA, T), lambda b: (0, b, 0)),
            pl.BlockSpec((DIN, H), lambda b: (0, 0)),
            pl.BlockSpec((1, H), lambda b: (0, 0)),
        ],
        out_specs=pl.BlockSpec((BBA, T, H), lambda b: (b, 0, 0)),
        out_shape=jax.ShapeDtypeStruct((B, T, H), jnp.bfloat16),
        compiler_params=pltpu.CompilerParams(
            dimension_semantics=("parallel",),
            vmem_limit_bytes=48 * 1024 * 1024,
        ),
        name="ctrnn_drive",
        interpret=interpret,
    )(xT, wx, bs)

    out_shape = (
        jax.ShapeDtypeStruct((B, T, H), jnp.float32),
        jax.ShapeDtypeStruct((B, T, DOUT), jnp.float32),
    )

    hstore, output = pl.pallas_call(
        partial(_scan_kernel, bb=BB, tt=TT, hdim=H),
        grid=(B // BB, T // TT),
        in_specs=[
            pl.BlockSpec((BB, TT, H), lambda b, t: (b, t, 0)),
            pl.BlockSpec((BB, TT, H), lambda b, t: (b, t, 0)),
            pl.BlockSpec((H, H), lambda b, t: (0, 0)),
            pl.BlockSpec((H, DOUT), lambda b, t: (0, 0)),
            pl.BlockSpec((1, H), lambda b, t: (0, 0)),
        ],
        out_specs=[
            pl.BlockSpec((BB, TT, H), lambda b, t: (b, t, 0)),
            pl.BlockSpec((BB, TT, DOUT), lambda b, t: (b, t, 0)),
        ],
        out_shape=out_shape,
        scratch_shapes=[
            pltpu.VMEM((BB, H), jnp.float32),
            pltpu.VMEM((BB, H), jnp.float32),
            pltpu.VMEM((BB, TT, H), jnp.float32),
        ],
        compiler_params=pltpu.CompilerParams(
            dimension_semantics=("parallel", "arbitrary"),
            vmem_limit_bytes=56 * 1024 * 1024,
        ),
        name="ctrnn_scan",
        interpret=interpret,
    )(drive, noise, wh, wy, ah0r)

    return output, hstore
